# baseline (speedup 1.0000x reference)
import jax
import jax.numpy as jnp
from jax.experimental import pallas as pl
from jax.experimental.pallas import tpu as pltpu

_CD = jnp.bfloat16


def _width_select():
    j = jnp.arange(14)
    iw = jnp.arange(4)
    src = (2 * j[None, :] + iw[:, None] - 1).reshape(1, 56)
    return (jnp.arange(28)[:, None] == src).astype(_CD)


def _conv1_blockdiag(w1):
    w4 = w1.reshape(3, 3, 32)
    eye = jnp.eye(14, dtype=w1.dtype)
    wbd = jnp.zeros((4, 4, 14, 14, 4, 32), w1.dtype)
    for ph in range(2):
        for pw in range(2):
            pp = 2 * ph + pw
            for dh in range(3):
                for dw in range(3):
                    blk = eye[:, :, None] * w4[dh, dw][None, None, :]
                    wbd = wbd.at[ph + dh, pw + dw, :, :, pp, :].set(blk)
    return wbd.reshape(224, 1792)


def _conv2_packed(w2):
    w4 = w2.reshape(3, 3, 32, 64)
    blocks = jnp.zeros((4, 4, 32, 4, 64), w2.dtype)
    for ph in range(2):
        for pw in range(2):
            pp = ph * 2 + pw
            blocks = blocks.at[ph:ph + 3, pw:pw + 3, :, pp, :].set(w4)
    return blocks.reshape(512, 256)


def _fused_cnn_kernel(x_ref, sw_ref, w1_ref, b1_ref, w2_ref, b2_ref,
                      wf1_ref, bf1_ref, wf2_ref, bf2_ref, o_ref):
    bT = x_ref.shape[0]

    x = x_ref[...].astype(_CD).reshape(bT * 28, 28)
    z = jnp.dot(x, sw_ref[...],
                preferred_element_type=jnp.float32).astype(_CD)
    z4 = z.reshape(bT, 14, 2, 56)
    ze = z4[:, :, 0, :]
    zo = z4[:, :, 1, :]
    zo_m1 = jnp.pad(zo, ((0, 0), (1, 0), (0, 0)))[:, :14, :]
    ze_p1 = jnp.pad(ze, ((0, 0), (0, 1), (0, 0)))[:, 1:, :]
    p1 = jnp.concatenate([zo_m1, ze, zo, ze_p1], axis=-1)
    a1 = jnp.dot(p1.reshape(bT * 14, 224), w1_ref[...],
                 preferred_element_type=jnp.float32).astype(_CD)
    a1 = a1.reshape(bT * 14, 14, 128)
    m1 = jnp.maximum(jnp.maximum(a1[..., 0:32], a1[..., 32:64]),
                     jnp.maximum(a1[..., 64:96], a1[..., 96:128]))
    y1 = jnp.maximum(m1 + b1_ref[...], jnp.bfloat16(0))
    y1 = y1.reshape(bT, 14, 14, 32)

    y1p = jnp.pad(y1, ((0, 0), (1, 1), (1, 1), (0, 0)))
    yq = y1p.reshape(bT, 8, 2, 8, 2, 32)
    planes2 = [[yq[:, :, a, :, b, :] for b in range(2)] for a in range(2)]
    taps2 = [planes2[ih % 2][iw % 2][:, ih // 2:ih // 2 + 7,
                                     iw // 2:iw // 2 + 7, :]
             for ih in range(4) for iw in range(4)]
    p2 = jnp.concatenate(taps2, axis=-1).reshape(bT * 49, 512)
    a2 = jnp.dot(p2, w2_ref[...],
                 preferred_element_type=jnp.float32).astype(_CD)
    a2 = a2.reshape(bT, 7, 7, 256)
    m2 = jnp.maximum(jnp.maximum(a2[..., 0:64], a2[..., 64:128]),
                     jnp.maximum(a2[..., 128:192], a2[..., 192:256]))
    y2 = jnp.maximum(m2 + b2_ref[...], jnp.bfloat16(0))
    flat = y2.reshape(bT, 7 * 7 * 64)

    h = jnp.dot(flat, wf1_ref[...], preferred_element_type=jnp.float32)
    h = jnp.maximum(h + bf1_ref[...], 0.0).astype(_CD)
    out = jnp.dot(h, wf2_ref[...], preferred_element_type=jnp.float32)
    o_ref[...] = out + bf2_ref[...]


def kernel(x_nchw, w1, b1, w2, b2, wfc1, bfc1, wfc2p, bfc2p):
    B = x_nchw.shape[0]
    x = x_nchw.reshape(B, 28, 28)

    sw = _width_select()
    w1bd = _conv1_blockdiag(w1)
    w2pk = _conv2_packed(w2)

    bT = 64
    while B % bT != 0:
        bT //= 2
    Np = wfc2p.shape[1]

    out = pl.pallas_call(
        _fused_cnn_kernel,
        out_shape=jax.ShapeDtypeStruct((B, Np), jnp.float32),
        grid=(B // bT,),
        in_specs=[
            pl.BlockSpec((bT, 28, 28), lambda i: (i, 0, 0)),
            pl.BlockSpec((28, 56), lambda i: (0, 0)),
            pl.BlockSpec((224, 1792), lambda i: (0, 0)),
            pl.BlockSpec((1, 32), lambda i: (0, 0)),
            pl.BlockSpec((512, 256), lambda i: (0, 0)),
            pl.BlockSpec((1, 64), lambda i: (0, 0)),
            pl.BlockSpec((3136, 256), lambda i: (0, 0)),
            pl.BlockSpec((1, 256), lambda i: (0, 0)),
            pl.BlockSpec((256, Np), lambda i: (0, 0)),
            pl.BlockSpec((1, Np), lambda i: (0, 0)),
        ],
        out_specs=pl.BlockSpec((bT, Np), lambda i: (i, 0)),
        compiler_params=pltpu.CompilerParams(
            dimension_semantics=("parallel",)),
    )(x, sw, w1bd, b1.astype(_CD).reshape(1, 32), w2pk,
      b2.astype(_CD).reshape(1, 64),
      wfc1, bfc1.reshape(1, 256), wfc2p, bfc2p.reshape(1, Np))
    return out[:, :10]


# speedup vs baseline: 1.9106x; 1.9106x over previous
import jax
import jax.numpy as jnp
from jax.experimental import pallas as pl
from jax.experimental.pallas import tpu as pltpu

_CD = jnp.bfloat16


def _width_select():
    j = jnp.arange(14)
    iw = jnp.arange(4)
    src = (2 * j[None, :] + iw[:, None] - 1).reshape(1, 56)
    return (jnp.arange(28)[:, None] == src).astype(_CD)


def _conv1_blockdiag(w1):
    w4 = w1.reshape(3, 3, 32)
    eye = jnp.eye(14, dtype=w1.dtype).reshape(14, 7, 2).transpose(0, 2, 1)
    wbd = jnp.zeros((4, 4, 14, 2, 7, 4, 32), w1.dtype)
    for ph in range(2):
        for pw in range(2):
            pp = 2 * ph + pw
            for dh in range(3):
                for dw in range(3):
                    blk = eye[:, :, :, None] * w4[dh, dw][None, None, None, :]
                    wbd = wbd.at[ph + dh, pw + dw, :, :, :, pp, :].set(blk)
    return wbd.reshape(224, 1792)


def _conv2_packed(w2):
    w4 = w2.reshape(3, 3, 32, 64)
    blocks = jnp.zeros((4, 4, 32, 4, 64), w2.dtype)
    for ph in range(2):
        for pw in range(2):
            pp = ph * 2 + pw
            blocks = blocks.at[ph:ph + 3, pw:pw + 3, :, pp, :].set(w4)
    return blocks.reshape(512, 256)


def _scatter_fc1(wfc1):
    w = jnp.zeros((8, 8, 64, 256), wfc1.dtype)
    w = w.at[:7, :7].set(wfc1.reshape(7, 7, 64, 256))
    return w.reshape(4096, 256)


def _fused_cnn_kernel(x_ref, sw_ref, w1_ref, b1_ref, w2_ref, b2_ref,
                      wf1_ref, bf1_ref, wf2_ref, bf2_ref, o_ref):
    bT = x_ref.shape[0]

    x = x_ref[...].astype(_CD).reshape(bT * 28, 28)
    z = jnp.dot(x, sw_ref[...],
                preferred_element_type=jnp.float32).astype(_CD)
    z4 = z.reshape(bT, 14, 2, 56)
    ze = z4[:, :, 0, :]
    zo = z4[:, :, 1, :]
    zo_m1 = jnp.pad(zo, ((0, 0), (1, 0), (0, 0)))[:, :14, :]
    ze_p1 = jnp.pad(ze, ((0, 0), (0, 1), (0, 0)))[:, 1:, :]
    p1 = jnp.concatenate([zo_m1, ze, zo, ze_p1], axis=-1)
    p1 = p1.reshape(bT, 7, 2, 224)
    pe = p1[:, :, 0, :].reshape(bT * 7, 224)
    po = p1[:, :, 1, :].reshape(bT * 7, 224)
    ys = []
    for ph in (pe, po):
        a = jnp.dot(ph, w1_ref[...],
                    preferred_element_type=jnp.float32)
        a = a.reshape(bT * 7, 2, 7, 128)
        m = jnp.maximum(jnp.maximum(a[..., 0:32], a[..., 32:64]),
                        jnp.maximum(a[..., 64:96], a[..., 96:128]))
        y = jnp.maximum(m + b1_ref[...], 0.0).astype(_CD)
        y = y.reshape(bT, 7, 2, 7, 32)
        ys.append([y[:, :, 0, :, :], y[:, :, 1, :, :]])

    def plane(a, b):
        q = ys[1 - a][1 - b]
        rp = (1, 0) if a == 0 else (0, 1)
        cp = (1, 0) if b == 0 else (0, 1)
        return jnp.pad(q, ((0, 0), rp, cp, (0, 0)))

    planes2 = [[plane(a, b) for b in range(2)] for a in range(2)]
    taps2 = []
    for ih in range(4):
        for iw in range(4):
            t = planes2[ih % 2][iw % 2]
            dh, dw = ih // 2, iw // 2
            if dh or dw:
                t = jnp.pad(t, ((0, 0), (0, dh), (0, dw), (0, 0))
                            )[:, dh:dh + 8, dw:dw + 8, :]
            taps2.append(t)
    p2 = jnp.concatenate(taps2, axis=-1).reshape(bT * 64, 512)
    a2 = jnp.dot(p2, w2_ref[...], preferred_element_type=jnp.float32)
    a2 = a2.reshape(bT, 8, 8, 256)
    m2 = jnp.maximum(jnp.maximum(a2[..., 0:64], a2[..., 64:128]),
                     jnp.maximum(a2[..., 128:192], a2[..., 192:256]))
    y2 = jnp.maximum(m2 + b2_ref[...], 0.0).astype(_CD)
    flat = y2.reshape(bT, 4096)

    h = jnp.dot(flat, wf1_ref[...], preferred_element_type=jnp.float32)
    h = jnp.maximum(h + bf1_ref[...], 0.0).astype(_CD)
    out = jnp.dot(h, wf2_ref[...], preferred_element_type=jnp.float32)
    o_ref[...] = out + bf2_ref[...]


def kernel(x_nchw, w1, b1, w2, b2, wfc1, bfc1, wfc2p, bfc2p):
    B = x_nchw.shape[0]
    x = x_nchw.reshape(B, 28, 28)

    sw = _width_select()
    w1bd = _conv1_blockdiag(w1)
    w2pk = _conv2_packed(w2)
    wf1s = _scatter_fc1(wfc1)

    bT = 64
    while B % bT != 0:
        bT //= 2
    Np = wfc2p.shape[1]

    out = pl.pallas_call(
        _fused_cnn_kernel,
        out_shape=jax.ShapeDtypeStruct((B, Np), jnp.float32),
        grid=(B // bT,),
        in_specs=[
            pl.BlockSpec((bT, 28, 28), lambda i: (i, 0, 0)),
            pl.BlockSpec((28, 56), lambda i: (0, 0)),
            pl.BlockSpec((224, 1792), lambda i: (0, 0)),
            pl.BlockSpec((1, 32), lambda i: (0, 0)),
            pl.BlockSpec((512, 256), lambda i: (0, 0)),
            pl.BlockSpec((1, 64), lambda i: (0, 0)),
            pl.BlockSpec((4096, 256), lambda i: (0, 0)),
            pl.BlockSpec((1, 256), lambda i: (0, 0)),
            pl.BlockSpec((256, Np), lambda i: (0, 0)),
            pl.BlockSpec((1, Np), lambda i: (0, 0)),
        ],
        out_specs=pl.BlockSpec((bT, Np), lambda i: (i, 0)),
        compiler_params=pltpu.CompilerParams(
            dimension_semantics=("parallel",)),
    )(x, sw, w1bd, b1.reshape(1, 32), w2pk, b2.reshape(1, 64),
      wf1s, bfc1.reshape(1, 256), wfc2p, bfc2p.reshape(1, Np))
    return out[:, :10]


# speedup vs baseline: 1.9643x; 1.0281x over previous
import jax
import jax.numpy as jnp
from jax.experimental import pallas as pl
from jax.experimental.pallas import tpu as pltpu

_CD = jnp.bfloat16


def _width_select():
    j = jnp.arange(14)
    iw = jnp.arange(4)
    src = (2 * j[None, :] + iw[:, None] - 1).reshape(1, 56)
    return (jnp.arange(28)[:, None] == src).astype(_CD)


def _conv1_blockdiag(w1):
    w4 = w1.reshape(3, 3, 32)
    eye = jnp.eye(14, dtype=w1.dtype).reshape(14, 7, 2).transpose(0, 2, 1)
    wbd = jnp.zeros((4, 4, 14, 2, 7, 4, 32), w1.dtype)
    for ph in range(2):
        for pw in range(2):
            pp = 2 * ph + pw
            for dh in range(3):
                for dw in range(3):
                    blk = eye[:, :, :, None] * w4[dh, dw][None, None, None, :]
                    wbd = wbd.at[ph + dh, pw + dw, :, :, :, pp, :].set(blk)
    return wbd.reshape(224, 1792)


def _conv2_packed(w2):
    w4 = w2.reshape(3, 3, 32, 64)
    blocks = jnp.zeros((4, 4, 32, 4, 64), w2.dtype)
    for ph in range(2):
        for pw in range(2):
            pp = ph * 2 + pw
            blocks = blocks.at[ph:ph + 3, pw:pw + 3, :, pp, :].set(w4)
    return blocks.reshape(512, 256)


def _scatter_fc1(wfc1):
    w = jnp.zeros((8, 8, 64, 256), wfc1.dtype)
    w = w.at[:7, :7].set(wfc1.reshape(7, 7, 64, 256))
    return w.reshape(4096, 256)


def _fused_cnn_kernel(x_ref, sw_ref, w1_ref, b1_ref, w2_ref, b2_ref,
                      wf1_ref, bf1_ref, wf2_ref, bf2_ref, o_ref):
    bT = x_ref.shape[0]

    x = x_ref[...].astype(_CD).reshape(bT * 28, 28)
    z = jnp.dot(x, sw_ref[...],
                preferred_element_type=jnp.float32).astype(_CD)
    z4 = z.reshape(bT, 14, 2, 56)
    ze = z4[:, :, 0, :]
    zo = z4[:, :, 1, :]
    zo_m1 = jnp.pad(zo, ((0, 0), (1, 0), (0, 0)))[:, :14, :]
    ze_p1 = jnp.pad(ze, ((0, 0), (0, 1), (0, 0)))[:, 1:, :]
    p1 = jnp.concatenate([zo_m1, ze, zo, ze_p1], axis=-1)
    p1 = p1.reshape(bT, 7, 2, 224)
    pe = p1[:, :, 0, :].reshape(bT * 7, 224)
    po = p1[:, :, 1, :].reshape(bT * 7, 224)
    ys = []
    for ph in (pe, po):
        a = jnp.dot(ph, w1_ref[...],
                    preferred_element_type=jnp.float32).astype(_CD)
        a = a.reshape(bT * 7, 2, 7, 128)
        m = jnp.maximum(jnp.maximum(a[..., 0:32], a[..., 32:64]),
                        jnp.maximum(a[..., 64:96], a[..., 96:128]))
        y = jnp.maximum(m + b1_ref[...], jnp.bfloat16(0))
        y = y.reshape(bT, 7, 2, 7, 32)
        ys.append([y[:, :, 0, :, :], y[:, :, 1, :, :]])

    def plane(a, b):
        q = ys[1 - a][1 - b]
        rp = (1, 0) if a == 0 else (0, 1)
        cp = (1, 0) if b == 0 else (0, 1)
        return jnp.pad(q, ((0, 0), rp, cp, (0, 0)))

    planes2 = [[plane(a, b) for b in range(2)] for a in range(2)]
    taps2 = []
    for ih in range(4):
        for iw in range(4):
            t = planes2[ih % 2][iw % 2]
            dh, dw = ih // 2, iw // 2
            if dh or dw:
                t = jnp.pad(t, ((0, 0), (0, dh), (0, dw), (0, 0))
                            )[:, dh:dh + 8, dw:dw + 8, :]
            taps2.append(t)
    p2 = jnp.concatenate(taps2, axis=-1).reshape(bT * 64, 512)
    a2 = jnp.dot(p2, w2_ref[...], preferred_element_type=jnp.float32)
    a2 = a2.reshape(bT, 8, 8, 256)
    m2 = jnp.maximum(jnp.maximum(a2[..., 0:64], a2[..., 64:128]),
                     jnp.maximum(a2[..., 128:192], a2[..., 192:256]))
    y2 = jnp.maximum(m2 + b2_ref[...], 0.0).astype(_CD)
    flat = y2.reshape(bT, 4096)

    h = jnp.dot(flat, wf1_ref[...], preferred_element_type=jnp.float32)
    h = jnp.maximum(h + bf1_ref[...], 0.0).astype(_CD)
    out = jnp.dot(h, wf2_ref[...], preferred_element_type=jnp.float32)
    o_ref[...] = out + bf2_ref[...]


def kernel(x_nchw, w1, b1, w2, b2, wfc1, bfc1, wfc2p, bfc2p):
    B = x_nchw.shape[0]
    x = x_nchw.reshape(B, 28, 28)

    sw = _width_select()
    w1bd = _conv1_blockdiag(w1)
    w2pk = _conv2_packed(w2)
    wf1s = _scatter_fc1(wfc1)

    bT = 64
    while B % bT != 0:
        bT //= 2
    Np = wfc2p.shape[1]

    out = pl.pallas_call(
        _fused_cnn_kernel,
        out_shape=jax.ShapeDtypeStruct((B, Np), jnp.float32),
        grid=(B // bT,),
        in_specs=[
            pl.BlockSpec((bT, 28, 28), lambda i: (i, 0, 0)),
            pl.BlockSpec((28, 56), lambda i: (0, 0)),
            pl.BlockSpec((224, 1792), lambda i: (0, 0)),
            pl.BlockSpec((1, 32), lambda i: (0, 0)),
            pl.BlockSpec((512, 256), lambda i: (0, 0)),
            pl.BlockSpec((1, 64), lambda i: (0, 0)),
            pl.BlockSpec((4096, 256), lambda i: (0, 0)),
            pl.BlockSpec((1, 256), lambda i: (0, 0)),
            pl.BlockSpec((256, Np), lambda i: (0, 0)),
            pl.BlockSpec((1, Np), lambda i: (0, 0)),
        ],
        out_specs=pl.BlockSpec((bT, Np), lambda i: (i, 0)),
        compiler_params=pltpu.CompilerParams(
            dimension_semantics=("parallel",)),
    )(x, sw, w1bd, b1.astype(_CD).reshape(1, 32), w2pk, b2.reshape(1, 64),
      wf1s, bfc1.reshape(1, 256), wfc2p, bfc2p.reshape(1, Np))
    return out[:, :10]
